# baseline (speedup 1.0000x reference)
import functools

import jax
import jax.numpy as jnp
from jax import lax
from jax.experimental import pallas as pl
from jax.experimental.pallas import tpu as pltpu
from jax.experimental.pallas import tpu_sc as plsc

N_E = 8192
E_DIM = 768
E_LAT = 256
N_TOK = 8192
TOK_B = 1024
JB = 512
N_JB = N_E // JB


def _k1_body(z_ref, wd_ref, bd_ref, emb_ref, zlt_out, idx_out,
             zlt2_sc, a_sc, rmin_sc, rblk_sc, bjs_sc, g2_sc):
    t = pl.program_id(0)
    j = pl.program_id(1)

    @pl.when(j == 0)
    def _():
        zb = z_ref[0]
        zl = lax.dot_general(wd_ref[...], zb, (((1,), (0,)), ((), ())),
                             preferred_element_type=jnp.float32)
        zl = zl + bd_ref[...]
        zlt_out[0] = zl
        a_sc[...] = jnp.sum(zl * zl, axis=0, keepdims=True)
        zlt2_sc[...] = zl + zl
        rmin_sc[...] = jnp.full((64, TOK_B), jnp.inf, jnp.float32)
        rblk_sc[...] = jnp.zeros((64, TOK_B), jnp.float32)

    @pl.when(jnp.logical_and(t == 0, j < N_JB))
    def _():
        eb0 = emb_ref[pl.ds(j * JB, JB)]
        bjs_sc[pl.ds(j * JB, JB)] = jnp.sum(eb0 * eb0, axis=1, keepdims=True)

    @pl.when(j < N_JB)
    def _():
        eb = emb_ref[pl.ds(j * JB, JB)]
        g2_sc[lax.bitwise_and(j, 1)] = lax.dot_general(
            eb, zlt2_sc[...], (((1,), (0,)), ((), ())),
            preferred_element_type=jnp.float32)

    @pl.when(j > 0)
    def _():
        jp = j - 1
        g2 = g2_sc[lax.bitwise_and(jp, 1)]
        a = a_sc[...]
        bj = bjs_sc[pl.ds(jp * JB, JB)]
        rmin = rmin_sc[...]
        rblk = rblk_sc[...]
        nch = JB // 64
        blk0 = (jp * nch).astype(jnp.float32)
        for r in range(nch):
            ds = (a + bj[64 * r:64 * r + 64]) - g2[64 * r:64 * r + 64]
            nmin = jnp.minimum(ds, rmin)
            upd = nmin != rmin
            rmin = nmin
            rblk = jnp.where(upd, blk0 + jnp.float32(r), rblk)
        rmin_sc[...] = rmin
        rblk_sc[...] = rblk

    @pl.when(j == pl.num_programs(1) - 1)
    def _():
        s_iota = lax.broadcasted_iota(jnp.int32, (64, TOK_B), 0).astype(
            jnp.float32)
        rid = rblk_sc[...] * 64.0 + s_iota
        v = rmin_sc[...]

        def merge(v0, i0, v1, i1):
            lt = (v1 < v0) | ((v1 == v0) & (i1 < i0))
            return jnp.where(lt, v1, v0), jnp.where(lt, i1, i0)

        n = 64
        while n > 1:
            h = n // 2
            v, rid = merge(v[0:h], rid[0:h], v[h:n], rid[h:n])
            n = h
        idx_out[0] = rid.astype(jnp.int32)


def _k1_call(z3, wd, bd2, emb, interpret=False):
    nb = z3.shape[0]
    return pl.pallas_call(
        _k1_body,
        grid=(nb, N_JB + 1),
        in_specs=[
            pl.BlockSpec((1, E_DIM, TOK_B), lambda t, j: (t, 0, 0)),
            pl.BlockSpec((E_LAT, E_DIM), lambda t, j: (0, 0)),
            pl.BlockSpec((E_LAT, 1), lambda t, j: (0, 0)),
            pl.BlockSpec((N_E, E_LAT), lambda t, j: (0, 0)),
        ],
        out_specs=[
            pl.BlockSpec((1, E_LAT, TOK_B), lambda t, j: (t, 0, 0)),
            pl.BlockSpec((1, 1, TOK_B), lambda t, j: (t, 0, 0)),
        ],
        out_shape=[
            jax.ShapeDtypeStruct((nb, E_LAT, TOK_B), jnp.float32),
            jax.ShapeDtypeStruct((nb, 1, TOK_B), jnp.int32),
        ],
        scratch_shapes=[
            pltpu.VMEM((E_LAT, TOK_B), jnp.float32),
            pltpu.VMEM((1, TOK_B), jnp.float32),
            pltpu.VMEM((64, TOK_B), jnp.float32),
            pltpu.VMEM((64, TOK_B), jnp.float32),
            pltpu.VMEM((N_E, 1), jnp.float32),
            pltpu.VMEM((2, JB, TOK_B), jnp.float32),
        ],
        compiler_params=pltpu.CompilerParams(
            dimension_semantics=("arbitrary", "arbitrary")),
        interpret=interpret,
    )(z3, wd, bd2, emb)


def _k3_body(zlt_ref, zql_ref, wu_ref, bu_ref, idx_ref,
             zq_out, loss_out, ppl_out, cu_out, acc_sc, cnt_sc):
    b = pl.program_id(0)
    nb = pl.num_programs(0)
    zl = zlt_ref[0]
    zqT = jnp.transpose(zql_ref[0], (1, 0))
    st = zl + (zqT - zl)
    zq = lax.dot_general(wu_ref[...], st, (((1,), (0,)), ((), ())),
                         preferred_element_type=jnp.float32) + bu_ref[...]
    zq_out[0] = zq
    diff = zqT - zl
    part = jnp.sum(diff * diff)

    idt = idx_ref[0]
    hi = lax.shift_right_logical(idt, 6)
    lo = lax.bitwise_and(idt, 63)
    hi_i = lax.broadcasted_iota(jnp.int32, (1, 128), 1)
    lo_i = lax.broadcasted_iota(jnp.int32, (1, 64), 1)
    m1 = (hi == hi_i).astype(jnp.float32)
    m2 = (lo == lo_i).astype(jnp.float32)
    pcnt = lax.dot_general(m1, m2, (((0,), (0,)), ((), ())),
                           preferred_element_type=jnp.float32)

    @pl.when(b == 0)
    def _():
        acc_sc[0] = part
        cnt_sc[...] = pcnt

    @pl.when(b > 0)
    def _():
        acc_sc[0] = acc_sc[0] + part
        cnt_sc[...] += pcnt

    @pl.when(b == nb - 1)
    def _():
        m = acc_sc[0] / jnp.float32(N_TOK * E_LAT)
        loss_out[0, 0] = m + 0.25 * m
        avg = cnt_sc[...] / jnp.float32(N_TOK)
        ent = jnp.sum(avg * jnp.log(avg + 1e-10))
        ppl_out[0, 0] = jnp.exp(-ent)
        cu_out[0, 0] = jnp.sum((avg > 0).astype(jnp.int32))


def _k3_call(zlt, zql, wu, bu2, idx3d, interpret=False):
    nb = zlt.shape[0]
    return pl.pallas_call(
        _k3_body,
        grid=(nb,),
        in_specs=[
            pl.BlockSpec((1, E_LAT, TOK_B), lambda b: (b, 0, 0)),
            pl.BlockSpec((1, TOK_B, E_LAT), lambda b: (b, 0, 0)),
            pl.BlockSpec((E_DIM, E_LAT), lambda b: (0, 0)),
            pl.BlockSpec((E_DIM, 1), lambda b: (0, 0)),
            pl.BlockSpec((1, TOK_B, 1), lambda b: (b, 0, 0)),
        ],
        out_specs=[
            pl.BlockSpec((1, E_DIM, TOK_B), lambda b: (b, 0, 0)),
            pl.BlockSpec(memory_space=pltpu.SMEM),
            pl.BlockSpec(memory_space=pltpu.SMEM),
            pl.BlockSpec(memory_space=pltpu.SMEM),
        ],
        out_shape=[
            jax.ShapeDtypeStruct((nb, E_DIM, TOK_B), jnp.float32),
            jax.ShapeDtypeStruct((1, 1), jnp.float32),
            jax.ShapeDtypeStruct((1, 1), jnp.float32),
            jax.ShapeDtypeStruct((1, 1), jnp.int32),
        ],
        scratch_shapes=[pltpu.SMEM((1,), jnp.float32),
                        pltpu.VMEM((128, 64), jnp.float32)],
        compiler_params=pltpu.CompilerParams(
            dimension_semantics=("arbitrary",)),
        interpret=interpret,
    )(zlt, zql, wu, bu2, idx3d)


def _sc_gather(emb, idx2d):
    mesh = plsc.VectorSubcoreMesh(core_axis_name="c", subcore_axis_name="s")

    @functools.partial(
        pl.kernel,
        mesh=mesh,
        out_type=jax.ShapeDtypeStruct((N_TOK, E_LAT), jnp.float32),
        scratch_types=[
            pltpu.VMEM((2, 128), jnp.int32),
            pltpu.VMEM((256, E_LAT), jnp.float32),
            pltpu.SemaphoreType.DMA,
        ],
    )
    def k2(emb_hbm, idx_hbm, out_hbm, idx_v, rows_v, sem):
        wid = lax.axis_index("s") * 2 + lax.axis_index("c")
        base = wid * 256
        pltpu.sync_copy(idx_hbm.at[pl.ds(wid * 2, 2)], idx_v)
        for k in range(2):
            pltpu.async_copy(emb_hbm.at[idx_v.at[k]],
                             rows_v.at[pl.ds(k * 128, 128)], sem).wait()
        pltpu.sync_copy(rows_v, out_hbm.at[pl.ds(base, 256)])

    return k2(emb, idx2d)


def kernel(z, proj_down_W, proj_down_b, proj_up_W, proj_up_b, embedding):
    nb = z.shape[0]
    z3 = z.reshape(nb, E_DIM, TOK_B)
    bd2 = proj_down_b.reshape(E_LAT, 1)
    bu2 = proj_up_b.reshape(E_DIM, 1)

    zlt, idx3 = _k1_call(z3, proj_down_W, bd2, embedding)
    idx2d = idx3.reshape(N_TOK // 128, 128)
    zql = _sc_gather(embedding, idx2d)
    zq3, loss, ppl, cu = _k3_call(zlt, zql.reshape(nb, TOK_B, E_LAT),
                                  proj_up_W, bu2,
                                  idx3.reshape(nb, TOK_B, 1))

    z_q = zq3.reshape(z.shape)
    return (z_q, loss.reshape(()), ppl.reshape(()), cu.reshape(()),
            idx3.reshape(N_TOK))


# speedup vs baseline: 1.2829x; 1.2829x over previous
import functools

import jax
import jax.numpy as jnp
from jax import lax
from jax.experimental import pallas as pl
from jax.experimental.pallas import tpu as pltpu
from jax.experimental.pallas import tpu_sc as plsc

N_E = 8192
E_DIM = 768
E_LAT = 256
N_TOK = 8192
TOK_B = 1024
NPAIR = 4
TOK_P = 2 * TOK_B
JB = 512
N_JB = N_E // JB
CH = 8


def _k1_body(z_ref, wd_ref, bd_ref, emb_ref, zlt_out, idx_out,
             zlt2_sc, a_sc, rmin_sc, rblk_sc, bjs_sc):
    t = pl.program_id(0)
    j = pl.program_id(1)

    @pl.when(j == 0)
    def _():
        for h in range(2):
            zh = z_ref[0, h]
            zl = lax.dot_general(wd_ref[...], zh, (((1,), (0,)), ((), ())),
                                 preferred_element_type=jnp.float32)
            zl = zl + bd_ref[...]
            zlt_out[0, :, pl.ds(h * TOK_B, TOK_B)] = zl
            a_sc[:, pl.ds(h * TOK_B, TOK_B)] = jnp.sum(zl * zl, axis=0,
                                                       keepdims=True)
            zlt2_sc[:, pl.ds(h * TOK_B, TOK_B)] = zl + zl
        rmin_sc[...] = jnp.full((CH, TOK_P), jnp.inf, jnp.float32)
        rblk_sc[...] = jnp.zeros((CH, TOK_P), jnp.float32)

    eb = emb_ref[pl.ds(j * JB, JB)]

    @pl.when(t == 0)
    def _():
        bjs_sc[pl.ds(j * JB, JB)] = jnp.sum(eb * eb, axis=1, keepdims=True)

    g2 = lax.dot_general(eb, zlt2_sc[...], (((1,), (0,)), ((), ())),
                         preferred_element_type=jnp.float32)
    a = a_sc[...]
    bj = bjs_sc[pl.ds(j * JB, JB)]
    rmin = rmin_sc[...]
    rblk = rblk_sc[...]
    nch = JB // CH
    for r in range(nch):
        ds = (a + bj[CH * r:CH * r + CH]) - g2[CH * r:CH * r + CH]
        nmin = jnp.minimum(ds, rmin)
        upd = nmin != rmin
        rmin = nmin
        rblk = jnp.where(upd, jnp.float32(j * nch + r), rblk)
    rmin_sc[...] = rmin
    rblk_sc[...] = rblk

    @pl.when(j == pl.num_programs(1) - 1)
    def _():
        s_iota = lax.broadcasted_iota(jnp.int32, (CH, TOK_P), 0).astype(
            jnp.float32)
        rid = rblk_sc[...] * jnp.float32(CH) + s_iota
        v = rmin_sc[...]

        def merge(v0, i0, v1, i1):
            lt = (v1 < v0) | ((v1 == v0) & (i1 < i0))
            return jnp.where(lt, v1, v0), jnp.where(lt, i1, i0)

        n = CH
        while n > 1:
            h = n // 2
            v, rid = merge(v[0:h], rid[0:h], v[h:n], rid[h:n])
            n = h
        ridi = rid.astype(jnp.int32)
        idx_out[0, 0] = ridi[:, 0:TOK_B]
        idx_out[0, 1] = ridi[:, TOK_B:TOK_P]


def _k1_call(z4, wd, bd2, emb, interpret=False):
    npair = z4.shape[0]
    return pl.pallas_call(
        _k1_body,
        grid=(npair, N_JB),
        in_specs=[
            pl.BlockSpec((1, 2, E_DIM, TOK_B), lambda t, j: (t, 0, 0, 0)),
            pl.BlockSpec((E_LAT, E_DIM), lambda t, j: (0, 0)),
            pl.BlockSpec((E_LAT, 1), lambda t, j: (0, 0)),
            pl.BlockSpec((N_E, E_LAT), lambda t, j: (0, 0)),
        ],
        out_specs=[
            pl.BlockSpec((1, E_LAT, TOK_P), lambda t, j: (t, 0, 0)),
            pl.BlockSpec((1, 2, 1, TOK_B), lambda t, j: (t, 0, 0, 0)),
        ],
        out_shape=[
            jax.ShapeDtypeStruct((npair, E_LAT, TOK_P), jnp.float32),
            jax.ShapeDtypeStruct((npair, 2, 1, TOK_B), jnp.int32),
        ],
        scratch_shapes=[
            pltpu.VMEM((E_LAT, TOK_P), jnp.float32),
            pltpu.VMEM((1, TOK_P), jnp.float32),
            pltpu.VMEM((CH, TOK_P), jnp.float32),
            pltpu.VMEM((CH, TOK_P), jnp.float32),
            pltpu.VMEM((N_E, 1), jnp.float32),
        ],
        compiler_params=pltpu.CompilerParams(
            dimension_semantics=("arbitrary", "arbitrary")),
        interpret=interpret,
    )(z4, wd, bd2, emb)


def _k3_body(zlt_ref, zql_ref, wu_ref, bu_ref, idx_ref,
             zq_out, loss_out, ppl_out, cu_out, acc_sc, cnt_sc):
    b = pl.program_id(0)
    nb = pl.num_programs(0)
    zl = zlt_ref[0]
    zqT = jnp.transpose(zql_ref[0], (1, 0))
    st = zl + (zqT - zl)
    zq = lax.dot_general(wu_ref[...], st, (((1,), (0,)), ((), ())),
                         preferred_element_type=jnp.float32) + bu_ref[...]
    zq_out[0, 0] = zq[:, 0:TOK_B]
    zq_out[0, 1] = zq[:, TOK_B:TOK_P]
    diff = zqT - zl
    part = jnp.sum(diff * diff)

    idt = idx_ref[0]
    hi = lax.shift_right_logical(idt, 6)
    lo = lax.bitwise_and(idt, 63)
    hi_i = lax.broadcasted_iota(jnp.int32, (1, 128), 1)
    lo_i = lax.broadcasted_iota(jnp.int32, (1, 64), 1)
    m1 = (hi == hi_i).astype(jnp.float32)
    m2 = (lo == lo_i).astype(jnp.float32)
    pcnt = lax.dot_general(m1, m2, (((0,), (0,)), ((), ())),
                           preferred_element_type=jnp.float32)

    @pl.when(b == 0)
    def _():
        acc_sc[0] = part
        cnt_sc[...] = pcnt

    @pl.when(b > 0)
    def _():
        acc_sc[0] = acc_sc[0] + part
        cnt_sc[...] += pcnt

    @pl.when(b == nb - 1)
    def _():
        m = acc_sc[0] / jnp.float32(N_TOK * E_LAT)
        loss_out[0, 0] = m + 0.25 * m
        avg = cnt_sc[...] / jnp.float32(N_TOK)
        ent = jnp.sum(avg * jnp.log(avg + 1e-10))
        ppl_out[0, 0] = jnp.exp(-ent)
        cu_out[0, 0] = jnp.sum((avg > 0).astype(jnp.int32))


def _k3_call(zlt, zql, wu, bu2, idx3d, interpret=False):
    npair = zlt.shape[0]
    return pl.pallas_call(
        _k3_body,
        grid=(npair,),
        in_specs=[
            pl.BlockSpec((1, E_LAT, TOK_P), lambda b: (b, 0, 0)),
            pl.BlockSpec((1, TOK_P, E_LAT), lambda b: (b, 0, 0)),
            pl.BlockSpec((E_DIM, E_LAT), lambda b: (0, 0)),
            pl.BlockSpec((E_DIM, 1), lambda b: (0, 0)),
            pl.BlockSpec((1, TOK_P, 1), lambda b: (b, 0, 0)),
        ],
        out_specs=[
            pl.BlockSpec((1, 2, E_DIM, TOK_B), lambda b: (b, 0, 0, 0)),
            pl.BlockSpec(memory_space=pltpu.SMEM),
            pl.BlockSpec(memory_space=pltpu.SMEM),
            pl.BlockSpec(memory_space=pltpu.SMEM),
        ],
        out_shape=[
            jax.ShapeDtypeStruct((npair, 2, E_DIM, TOK_B), jnp.float32),
            jax.ShapeDtypeStruct((1, 1), jnp.float32),
            jax.ShapeDtypeStruct((1, 1), jnp.float32),
            jax.ShapeDtypeStruct((1, 1), jnp.int32),
        ],
        scratch_shapes=[pltpu.SMEM((1,), jnp.float32),
                        pltpu.VMEM((128, 64), jnp.float32)],
        compiler_params=pltpu.CompilerParams(
            dimension_semantics=("arbitrary",)),
        interpret=interpret,
    )(zlt, zql, wu, bu2, idx3d)


def _sc_gather(emb, idx2d):
    mesh = plsc.VectorSubcoreMesh(core_axis_name="c", subcore_axis_name="s")

    @functools.partial(
        pl.kernel,
        mesh=mesh,
        out_type=jax.ShapeDtypeStruct((N_TOK, E_LAT), jnp.float32),
        scratch_types=[
            pltpu.VMEM((2, 128), jnp.int32),
            pltpu.VMEM((256, E_LAT), jnp.float32),
            pltpu.SemaphoreType.DMA,
        ],
    )
    def k2(emb_hbm, idx_hbm, out_hbm, idx_v, rows_v, sem):
        wid = lax.axis_index("s") * 2 + lax.axis_index("c")
        base = wid * 256
        pltpu.sync_copy(idx_hbm.at[pl.ds(wid * 2, 2)], idx_v)
        for k in range(2):
            pltpu.async_copy(emb_hbm.at[idx_v.at[k]],
                             rows_v.at[pl.ds(k * 128, 128)], sem).wait()
        pltpu.sync_copy(rows_v, out_hbm.at[pl.ds(base, 256)])

    return k2(emb, idx2d)


def kernel(z, proj_down_W, proj_down_b, proj_up_W, proj_up_b, embedding):
    nb = z.shape[0]
    z4 = z.reshape(nb // 2, 2, E_DIM, TOK_B)
    bd2 = proj_down_b.reshape(E_LAT, 1)
    bu2 = proj_up_b.reshape(E_DIM, 1)

    zlt, idx4 = _k1_call(z4, proj_down_W, bd2, embedding)
    idxf = idx4.reshape(N_TOK)
    idx2d = idxf.reshape(N_TOK // 128, 128)
    zql = _sc_gather(embedding, idx2d)
    zq4, loss, ppl, cu = _k3_call(zlt, zql.reshape(nb // 2, TOK_P, E_LAT),
                                  proj_up_W, bu2,
                                  idxf.reshape(nb // 2, TOK_P, 1))

    z_q = zq4.reshape(z.shape)
    return (z_q, loss.reshape(()), ppl.reshape(()), cu.reshape(()), idxf)


# speedup vs baseline: 1.3408x; 1.0452x over previous
import functools

import jax
import jax.numpy as jnp
from jax import lax
from jax.experimental import pallas as pl
from jax.experimental.pallas import tpu as pltpu
from jax.experimental.pallas import tpu_sc as plsc

N_E = 8192
E_DIM = 768
E_LAT = 256
N_TOK = 8192
TOK_B = 1024
GRP = 4
TOK_G = GRP * TOK_B
TOK_P = 2 * TOK_B
JB = 512
N_JB = N_E // JB
CH = 8


def _k1_body(z_ref, wd_ref, bd_ref, emb_ref, zlt_out, idx_out,
             zlt2_sc, a_sc, rmin_sc, rblk_sc, bjs_sc):
    t = pl.program_id(0)
    j = pl.program_id(1)

    @pl.when(j == 0)
    def _():
        for h in range(GRP):
            zh = z_ref[0, h]
            zl = lax.dot_general(wd_ref[...], zh, (((1,), (0,)), ((), ())),
                                 preferred_element_type=jnp.float32)
            zl = zl + bd_ref[...]
            zlt_out[0, h // 2, :, pl.ds((h % 2) * TOK_B, TOK_B)] = zl
            a_sc[:, pl.ds(h * TOK_B, TOK_B)] = jnp.sum(zl * zl, axis=0,
                                                       keepdims=True)
            zlt2_sc[:, pl.ds(h * TOK_B, TOK_B)] = zl + zl
        rmin_sc[...] = jnp.full((CH, TOK_G), jnp.inf, jnp.float32)
        rblk_sc[...] = jnp.zeros((CH, TOK_G), jnp.float32)

    eb = emb_ref[pl.ds(j * JB, JB)]

    @pl.when(t == 0)
    def _():
        bjs_sc[pl.ds(j * JB, JB)] = jnp.sum(eb * eb, axis=1, keepdims=True)

    g2 = lax.dot_general(eb, zlt2_sc[...], (((1,), (0,)), ((), ())),
                         preferred_element_type=jnp.float32)
    a = a_sc[...]
    bj = bjs_sc[pl.ds(j * JB, JB)]
    rmin = rmin_sc[...]
    rblk = rblk_sc[...]
    nch = JB // CH
    for r in range(nch):
        ds = (a + bj[CH * r:CH * r + CH]) - g2[CH * r:CH * r + CH]
        nmin = jnp.minimum(ds, rmin)
        upd = nmin != rmin
        rmin = nmin
        rblk = jnp.where(upd, jnp.float32(j * nch + r), rblk)
    rmin_sc[...] = rmin
    rblk_sc[...] = rblk

    @pl.when(j == pl.num_programs(1) - 1)
    def _():
        s_iota = lax.broadcasted_iota(jnp.int32, (CH, TOK_G), 0).astype(
            jnp.float32)
        rid = rblk_sc[...] * jnp.float32(CH) + s_iota
        v = rmin_sc[...]

        def merge(v0, i0, v1, i1):
            lt = (v1 < v0) | ((v1 == v0) & (i1 < i0))
            return jnp.where(lt, v1, v0), jnp.where(lt, i1, i0)

        n = CH
        while n > 1:
            h = n // 2
            v, rid = merge(v[0:h], rid[0:h], v[h:n], rid[h:n])
            n = h
        ridi = rid.astype(jnp.int32)
        for h in range(GRP):
            idx_out[0, h] = ridi[:, h * TOK_B:(h + 1) * TOK_B]


def _k1_call(z4, wd, bd2, emb, interpret=False):
    ngrp = z4.shape[0]
    return pl.pallas_call(
        _k1_body,
        grid=(ngrp, N_JB),
        in_specs=[
            pl.BlockSpec((1, GRP, E_DIM, TOK_B), lambda t, j: (t, 0, 0, 0)),
            pl.BlockSpec((E_LAT, E_DIM), lambda t, j: (0, 0)),
            pl.BlockSpec((E_LAT, 1), lambda t, j: (0, 0)),
            pl.BlockSpec((N_E, E_LAT), lambda t, j: (0, 0)),
        ],
        out_specs=[
            pl.BlockSpec((1, GRP // 2, E_LAT, TOK_P),
                         lambda t, j: (t, 0, 0, 0)),
            pl.BlockSpec((1, GRP, 1, TOK_B), lambda t, j: (t, 0, 0, 0)),
        ],
        out_shape=[
            jax.ShapeDtypeStruct((ngrp, GRP // 2, E_LAT, TOK_P),
                                 jnp.float32),
            jax.ShapeDtypeStruct((ngrp, GRP, 1, TOK_B), jnp.int32),
        ],
        scratch_shapes=[
            pltpu.VMEM((E_LAT, TOK_G), jnp.float32),
            pltpu.VMEM((1, TOK_G), jnp.float32),
            pltpu.VMEM((CH, TOK_G), jnp.float32),
            pltpu.VMEM((CH, TOK_G), jnp.float32),
            pltpu.VMEM((N_E, 1), jnp.float32),
        ],
        compiler_params=pltpu.CompilerParams(
            dimension_semantics=("arbitrary", "arbitrary")),
        interpret=interpret,
    )(z4, wd, bd2, emb)


def _k3_body(zlt_ref, zql_ref, wu_ref, bu_ref, idx_ref,
             zq_out, loss_out, ppl_out, cu_out, acc_sc, cnt_sc):
    b = pl.program_id(0)
    nb = pl.num_programs(0)
    zl = zlt_ref[0]
    zqT = jnp.transpose(zql_ref[0], (1, 0))
    st = zl + (zqT - zl)
    zq = lax.dot_general(wu_ref[...], st, (((1,), (0,)), ((), ())),
                         preferred_element_type=jnp.float32) + bu_ref[...]
    zq_out[0, 0] = zq[:, 0:TOK_B]
    zq_out[0, 1] = zq[:, TOK_B:TOK_P]
    diff = zqT - zl
    part = jnp.sum(diff * diff)

    idt = idx_ref[0]
    hi = lax.shift_right_logical(idt, 6)
    lo = lax.bitwise_and(idt, 63)
    hi_i = lax.broadcasted_iota(jnp.int32, (1, 128), 1)
    lo_i = lax.broadcasted_iota(jnp.int32, (1, 64), 1)
    m1 = (hi == hi_i).astype(jnp.float32)
    m2 = (lo == lo_i).astype(jnp.float32)
    pcnt = lax.dot_general(m1, m2, (((0,), (0,)), ((), ())),
                           preferred_element_type=jnp.float32)

    @pl.when(b == 0)
    def _():
        acc_sc[0] = part
        cnt_sc[...] = pcnt

    @pl.when(b > 0)
    def _():
        acc_sc[0] = acc_sc[0] + part
        cnt_sc[...] += pcnt

    @pl.when(b == nb - 1)
    def _():
        m = acc_sc[0] / jnp.float32(N_TOK * E_LAT)
        loss_out[0, 0] = m + 0.25 * m
        avg = cnt_sc[...] / jnp.float32(N_TOK)
        ent = jnp.sum(avg * jnp.log(avg + 1e-10))
        ppl_out[0, 0] = jnp.exp(-ent)
        cu_out[0, 0] = jnp.sum((avg > 0).astype(jnp.int32))


def _k3_call(zlt, zql, wu, bu2, idx3d, interpret=False):
    npair = zlt.shape[0]
    return pl.pallas_call(
        _k3_body,
        grid=(npair,),
        in_specs=[
            pl.BlockSpec((1, E_LAT, TOK_P), lambda b: (b, 0, 0)),
            pl.BlockSpec((1, TOK_P, E_LAT), lambda b: (b, 0, 0)),
            pl.BlockSpec((E_DIM, E_LAT), lambda b: (0, 0)),
            pl.BlockSpec((E_DIM, 1), lambda b: (0, 0)),
            pl.BlockSpec((1, TOK_P, 1), lambda b: (b, 0, 0)),
        ],
        out_specs=[
            pl.BlockSpec((1, 2, E_DIM, TOK_B), lambda b: (b, 0, 0, 0)),
            pl.BlockSpec(memory_space=pltpu.SMEM),
            pl.BlockSpec(memory_space=pltpu.SMEM),
            pl.BlockSpec(memory_space=pltpu.SMEM),
        ],
        out_shape=[
            jax.ShapeDtypeStruct((npair, 2, E_DIM, TOK_B), jnp.float32),
            jax.ShapeDtypeStruct((1, 1), jnp.float32),
            jax.ShapeDtypeStruct((1, 1), jnp.float32),
            jax.ShapeDtypeStruct((1, 1), jnp.int32),
        ],
        scratch_shapes=[pltpu.SMEM((1,), jnp.float32),
                        pltpu.VMEM((128, 64), jnp.float32)],
        compiler_params=pltpu.CompilerParams(
            dimension_semantics=("arbitrary",)),
        interpret=interpret,
    )(zlt, zql, wu, bu2, idx3d)


def _sc_gather(emb, idx2d):
    mesh = plsc.VectorSubcoreMesh(core_axis_name="c", subcore_axis_name="s")

    @functools.partial(
        pl.kernel,
        mesh=mesh,
        out_type=jax.ShapeDtypeStruct((N_TOK, E_LAT), jnp.float32),
        scratch_types=[
            pltpu.VMEM((2, 128), jnp.int32),
            pltpu.VMEM((256, E_LAT), jnp.float32),
            pltpu.SemaphoreType.DMA,
        ],
    )
    def k2(emb_hbm, idx_hbm, out_hbm, idx_v, rows_v, sem):
        wid = lax.axis_index("s") * 2 + lax.axis_index("c")
        base = wid * 256
        pltpu.sync_copy(idx_hbm.at[pl.ds(wid * 2, 2)], idx_v)
        for k in range(2):
            pltpu.async_copy(emb_hbm.at[idx_v.at[k]],
                             rows_v.at[pl.ds(k * 128, 128)], sem).wait()
        pltpu.sync_copy(rows_v, out_hbm.at[pl.ds(base, 256)])

    return k2(emb, idx2d)


def kernel(z, proj_down_W, proj_down_b, proj_up_W, proj_up_b, embedding):
    nb = z.shape[0]
    z4 = z.reshape(nb // GRP, GRP, E_DIM, TOK_B)
    bd2 = proj_down_b.reshape(E_LAT, 1)
    bu2 = proj_up_b.reshape(E_DIM, 1)

    zlt, idx4 = _k1_call(z4, proj_down_W, bd2, embedding)
    idxf = idx4.reshape(N_TOK)
    idx2d = idxf.reshape(N_TOK // 128, 128)
    zql = _sc_gather(embedding, idx2d)
    zq4, loss, ppl, cu = _k3_call(zlt.reshape(nb // 2, E_LAT, TOK_P),
                                  zql.reshape(nb // 2, TOK_P, E_LAT),
                                  proj_up_W, bu2,
                                  idxf.reshape(nb // 2, TOK_P, 1))

    z_q = zq4.reshape(z.shape)
    return (z_q, loss.reshape(()), ppl.reshape(()), cu.reshape(()), idxf)
